# baseline (speedup 1.0000x reference)
import functools

import jax
import jax.numpy as jnp
from jax import lax
from jax.experimental import pallas as pl
from jax.experimental.pallas import tpu as pltpu
from jax.experimental.pallas import tpu_sc as plsc

NC = 2
NS = 16
LANES = 16

K = 80


def _sc_accumulate(n_nodes, n_hedge_feat, e_node, e_hedge,
                   node_features, hedge_features,
                   node_idx, h2n_idx):
  N = n_nodes
  DIN = node_features.shape[1]
  DH = n_hedge_feat
  nblk = e_node // K
  bpt = nblk // (NC * NS)
  assert bpt * NC * NS == nblk and nblk * K == e_node
  assert e_hedge == e_node
  SB = 10
  nsb = bpt // SB
  assert bpt == nsb * SB + 5
  G = N // 8
  ZC = 10

  mesh = plsc.VectorSubcoreMesh(core_axis_name="c", subcore_axis_name="s",
                                num_cores=NC, num_subcores=NS)

  @functools.partial(
      pl.kernel,
      out_type=[
          jax.ShapeDtypeStruct((NC * N, DIN), jnp.float32),
          jax.ShapeDtypeStruct((NC * N, 16), jnp.float32),
          jax.ShapeDtypeStruct((NC * N, DH), jnp.float32),
      ],
      mesh=mesh,
      scratch_types=[
          pltpu.VMEM((SB, 3, K), jnp.int32),
          pltpu.VMEM((K, DIN), jnp.float32),
          pltpu.VMEM((K, DIN), jnp.float32),
          pltpu.VMEM((K, 16), jnp.float32),
          pltpu.VMEM((K, 16), jnp.float32),
          pltpu.VMEM((K, DH), jnp.float32),
          pltpu.VMEM((K, DH), jnp.float32),
          pltpu.VMEM_SHARED((N, DIN), jnp.float32),
          pltpu.VMEM_SHARED((N, 16), jnp.float32),
          pltpu.VMEM_SHARED((N, DH), jnp.float32),
          pltpu.SemaphoreType.DMA,
          pltpu.SemaphoreType.DMA,
          pltpu.SemaphoreType.DMA,
          pltpu.SemaphoreType.DMA,
      ],
      compiler_params=pltpu.CompilerParams(use_tc_tiling_on_sc=False,
                                           needs_layout_passes=False),
  )
  def sc_kernel(nf_hbm, hf_hbm, nidx_hbm, hidx_hbm,
                p_msg, p_cnt, p_hdg,
                ib, rows0, rows1, cnt0, cnt1, hrows0, hrows1,
                acc, accc, acc2,
                semg0, semg1, semw0, semw1):
    c = lax.axis_index("c")
    s = lax.axis_index("s")
    zero16 = jnp.zeros((LANES,), jnp.float32)
    lane_iota = lax.broadcasted_iota(jnp.int32, (LANES,), 0)
    lane0 = lane_iota == 0
    lane1 = lane_iota == 1

    def splat(vec, l):
      return vec.at[jnp.full((LANES,), l, jnp.int32)].get(
          mode="promise_in_bounds")

    def zrows_body(i, _):
      r = i // (DIN // LANES)
      j = i % (DIN // LANES)
      rows0[r, pl.ds(j * LANES, LANES)] = zero16
      return 0
    lax.fori_loop(0, K * (DIN // LANES), zrows_body, 0)

    def zcnt_body(i, _):
      cnt0[i, :] = zero16
      return 0
    lax.fori_loop(0, K, zcnt_body, 0)

    g_lo = (s * G) // NS
    g_hi = ((s + 1) * G) // NS
    nchunk = (g_hi - g_lo) // ZC

    def zchunk_body(k, _):
      r = (g_lo + k * ZC) * 8
      pltpu.sync_copy(rows0, acc.at[pl.ds(r, ZC * 8)])
      pltpu.sync_copy(cnt0, accc.at[pl.ds(r, ZC * 8)])
      pltpu.sync_copy(cnt0, acc2.at[pl.ds(r, ZC * 8)])
      return 0
    lax.fori_loop(0, nchunk, zchunk_body, 0)

    def zrem_body(g, _):
      r = g * 8
      pltpu.sync_copy(rows0.at[pl.ds(0, 8)], acc.at[pl.ds(r, 8)])
      pltpu.sync_copy(cnt0.at[pl.ds(0, 8)], accc.at[pl.ds(r, 8)])
      pltpu.sync_copy(cnt0.at[pl.ds(0, 8)], acc2.at[pl.ds(r, 8)])
      return 0
    lax.fori_loop(g_lo + nchunk * ZC, g_hi, zrem_body, 0)
    plsc.subcore_barrier()

    b_base = (c * NS + s) * bpt

    def make_phase(feat, idx_hbm, rbufs, mask, accd):
      dw = rbufs[0].shape[1]
      cnts = (cnt0, cnt1)
      semgs, semws = (semg0, semg1), (semw0, semw1)

      def stage(t0, n):
        pltpu.sync_copy(idx_hbm.at[pl.ds(b_base + t0, n)],
                        ib.at[pl.ds(0, n)])

      def gather(r, u):
        pltpu.async_copy(feat.at[ib.at[r, 0]], rbufs[u], semgs[u])

      def wait_g(u):
        pltpu.make_async_copy(feat.at[ib.at[0, 0]], rbufs[u],
                              semgs[u]).wait()

      def scale(r, u):
        rows = rbufs[u]
        cnt = cnts[u]

        def grp(kk, _):
          cvec = plsc.bitcast(ib[r, 2, pl.ds(kk * LANES, LANES)],
                              jnp.float32)
          for l in range(LANES):
            cs = splat(cvec, l)
            e = kk * LANES + l
            cnt[e, :] = jnp.where(mask, cs, 0.0)
            for j in range(dw // LANES):
              rows[e, pl.ds(j * LANES, LANES)] = (
                  rows[e, pl.ds(j * LANES, LANES)] * cs)
          return 0
        lax.fori_loop(0, K // LANES, grp, 0)

      def scatter(r, u):
        pltpu.async_copy(rbufs[u], accd.at[ib.at[r, 1]], semws[u], add=True)
        pltpu.async_copy(cnts[u], accc.at[ib.at[r, 1]], semws[u], add=True)

      def wait_w(u):
        pltpu.make_async_copy(rbufs[u], accd.at[ib.at[0, 1]],
                              semws[u]).wait()
        pltpu.make_async_copy(cnts[u], accc.at[ib.at[0, 1]],
                              semws[u]).wait()

      stage(0, SB)
      gather(0, 0)
      gather(1, 1)

      def sb_body(t, _):
        for p in range(SB // 2):
          r0, r1 = 2 * p, 2 * p + 1
          wait_g(0)
          scale(r0, 0)
          scatter(r0, 0)
          wait_g(1)
          scale(r1, 1)
          scatter(r1, 1)
          if p < SB // 2 - 1:
            wait_w(0)
            gather(r0 + 2, 0)
            wait_w(1)
            gather(r1 + 2, 1)
          else:
            wait_w(0)
            wait_w(1)

            @pl.when(t < nsb - 1)
            def _():
              stage((t + 1) * SB, SB)
              gather(0, 0)
              gather(1, 1)

            @pl.when(t == nsb - 1)
            def _():
              stage(nsb * SB, 5)
              gather(0, 0)
              gather(1, 1)
        return 0
      lax.fori_loop(0, nsb, sb_body, 0)

      wait_g(0)
      scale(0, 0)
      scatter(0, 0)
      wait_g(1)
      scale(1, 1)
      scatter(1, 1)
      wait_w(0)
      gather(2, 0)
      wait_w(1)
      gather(3, 1)
      wait_g(0)
      scale(2, 0)
      scatter(2, 0)
      wait_g(1)
      scale(3, 1)
      scatter(3, 1)
      wait_w(0)
      gather(4, 0)
      wait_w(1)
      wait_g(0)
      scale(4, 0)
      scatter(4, 0)
      wait_w(0)

    make_phase(nf_hbm, nidx_hbm, (rows0, rows1), lane0, acc)
    make_phase(hf_hbm, hidx_hbm, (hrows0, hrows1), lane1, acc2)

    plsc.subcore_barrier()

    def ochunk_body(k, _):
      r = (g_lo + k * ZC) * 8
      o = c * N + r
      pltpu.sync_copy(acc.at[pl.ds(r, ZC * 8)], p_msg.at[pl.ds(o, ZC * 8)])
      pltpu.sync_copy(accc.at[pl.ds(r, ZC * 8)], p_cnt.at[pl.ds(o, ZC * 8)])
      pltpu.sync_copy(acc2.at[pl.ds(r, ZC * 8)], p_hdg.at[pl.ds(o, ZC * 8)])
      return 0
    lax.fori_loop(0, nchunk, ochunk_body, 0)

    def orem_body(g, _):
      r = g * 8
      o = c * N + r
      pltpu.sync_copy(acc.at[pl.ds(r, 8)], p_msg.at[pl.ds(o, 8)])
      pltpu.sync_copy(accc.at[pl.ds(r, 8)], p_cnt.at[pl.ds(o, 8)])
      pltpu.sync_copy(acc2.at[pl.ds(r, 8)], p_hdg.at[pl.ds(o, 8)])
      return 0
    lax.fori_loop(g_lo + nchunk * ZC, g_hi, orem_body, 0)

  return sc_kernel(node_features, hedge_features, node_idx, h2n_idx)


def _tc_combine(p_msg, p_cnt, p_hdg, W_msg, b_msg, W_scale, b_scale):
  N2, DIN = p_msg.shape
  N = N2 // NC
  DH = p_hdg.shape[1]
  DOUT = W_msg.shape[0]
  R = 1000
  nblk = N // R

  def body(m0, m1, c0, c1, h0, h1, wm, bm, ws, bs, out):
    a = m0[...] + m1[...]
    cnt = c0[...][:, 0:2] + c1[...][:, 0:2]
    hdg = h0[...] + h1[...]
    msg = lax.dot_general(a, wm[...], (((1,), (1,)), ((), ())),
                          precision=lax.Precision.HIGHEST,
                          preferred_element_type=jnp.float32)
    msg = msg + cnt[:, 0:1] * bm[...]
    scl = lax.dot_general(hdg, ws[...], (((1,), (1,)), ((), ())),
                          precision=lax.Precision.HIGHEST,
                          preferred_element_type=jnp.float32)
    scl = scl + cnt[:, 1:2] * bs[...]
    out[...] = jnp.tanh(scl * msg)

  return pl.pallas_call(
      body,
      grid=(nblk,),
      in_specs=[
          pl.BlockSpec((R, DIN), lambda i: (i, 0)),
          pl.BlockSpec((R, DIN), lambda i: (i + nblk, 0)),
          pl.BlockSpec((R, 16), lambda i: (i, 0)),
          pl.BlockSpec((R, 16), lambda i: (i + nblk, 0)),
          pl.BlockSpec((R, DH), lambda i: (i, 0)),
          pl.BlockSpec((R, DH), lambda i: (i + nblk, 0)),
          pl.BlockSpec((DOUT, DIN), lambda i: (0, 0)),
          pl.BlockSpec((1, DOUT), lambda i: (0, 0)),
          pl.BlockSpec((DOUT, DH), lambda i: (0, 0)),
          pl.BlockSpec((1, DOUT), lambda i: (0, 0)),
      ],
      out_specs=pl.BlockSpec((R, DOUT), lambda i: (i, 0)),
      out_shape=jax.ShapeDtypeStruct((N, DOUT), jnp.float32),
  )(p_msg, p_msg, p_cnt, p_cnt, p_hdg, p_hdg,
    W_msg, b_msg.reshape(1, DOUT), W_scale, b_scale.reshape(1, DOUT))


@jax.jit
def kernel(node_features, hedge_features, node_senders, node_receivers,
           node_convolution, hedge2node_senders, hedge2node_receivers,
           hedge2node_convolution, W_msg, b_msg, W_scale, b_scale):
  N, DIN = node_features.shape
  DH = hedge_features.shape[1]
  E = node_senders.shape[0]
  E2 = hedge2node_senders.shape[0]
  nblk = E // K

  def pack_idx(snd, rcv, cnv):
    return jnp.concatenate([
        snd.reshape(nblk, 1, K),
        rcv.reshape(nblk, 1, K),
        lax.bitcast_convert_type(cnv.reshape(nblk, 1, K), jnp.int32),
    ], axis=1)

  node_idx = pack_idx(node_senders, node_receivers,
                      node_convolution.reshape(E))
  h2n_idx = pack_idx(hedge2node_senders, hedge2node_receivers,
                     hedge2node_convolution.reshape(E2))

  p_msg, p_cnt, p_hdg = _sc_accumulate(
      N, DH, E, E2, node_features, hedge_features, node_idx, h2n_idx)

  return _tc_combine(p_msg, p_cnt, p_hdg, W_msg, b_msg, W_scale, b_scale)
